# Initial kernel scaffold; baseline (speedup 1.0000x reference)
import jax
import jax.numpy as jnp
from jax.experimental import pallas as pl


def kernel(x_list, Wg, bg, W1, b1, W2, b2, Wf, bf):
    raise NotImplementedError("write your pallas kernel here")


# speedup vs baseline: 1.1113x; 1.1113x over previous
import functools

import jax
import jax.numpy as jnp
from jax import lax
from jax.experimental import pallas as pl
from jax.experimental.pallas import tpu as pltpu

B = 64
D = 1024
HID = 1024
OUT = 1024
E = 64
K = 2


def _router_body(x_ref, wg_ref, bg_ref, probs_ref, g_ref, idx_ref):
    h = jnp.dot(x_ref[...], wg_ref[...], preferred_element_type=jnp.float32)
    h = h + bg_ref[...]
    m1 = jnp.max(h, axis=1, keepdims=True)
    e_all = jnp.exp(h - m1)
    probs_ref[...] = e_all / jnp.sum(e_all, axis=1, keepdims=True)

    is_max = h == m1
    cnt = jnp.sum(is_max.astype(jnp.float32), axis=1, keepdims=True)
    m2 = jnp.max(jnp.where(is_max, -jnp.inf, h), axis=1, keepdims=True)
    kth = jnp.where(cnt >= 2.0, m1, m2)
    mask = h >= kth
    gnum = jnp.where(mask, e_all, 0.0)
    g_ref[...] = gnum / jnp.sum(gnum, axis=1, keepdims=True)

    active = jnp.max(mask.astype(jnp.float32), axis=0, keepdims=True)
    lt = (lax.broadcasted_iota(jnp.int32, (E, E), 0)
          <= lax.broadcasted_iota(jnp.int32, (E, E), 1)).astype(jnp.float32)
    c_row = jnp.dot(active, lt)
    count = jnp.sum(active)
    iota_row = lax.broadcasted_iota(jnp.int32, (1, E), 1).astype(jnp.float32)
    last = jnp.max(jnp.where(active > 0.0, iota_row, -1.0))
    j_sub = lax.broadcasted_iota(jnp.int32, (E, E), 0).astype(jnp.float32)
    e_lane = lax.broadcasted_iota(jnp.int32, (E, E), 1).astype(jnp.float32)
    slot = (c_row - 1.0 == j_sub) & (active > 0.0)
    idx_col = jnp.sum(jnp.where(slot, e_lane, 0.0), axis=1, keepdims=True)
    j_col = lax.broadcasted_iota(jnp.int32, (E, 1), 0).astype(jnp.float32)
    idx_ref[...] = jnp.where(j_col < count, idx_col, last).astype(jnp.int32)


def _expert_body(idx_ref, x_ref, g_ref, w1_ref, b1_ref, w2_ref, b2_ref,
                 wf_ref, bf_ref, out_ref, acc_ref, xb_ref):
    i = pl.program_id(0)
    e = idx_ref[i]
    prev = idx_ref[jnp.maximum(i - 1, 0)]
    is_new = (i == 0) | (e != prev)

    @pl.when(i == 0)
    def _init():
        acc_ref[...] = jnp.zeros_like(acc_ref)
        xb_ref[...] = x_ref[...].astype(jnp.bfloat16)

    @pl.when(is_new)
    def _compute():
        w1 = w1_ref[0].astype(jnp.bfloat16)
        h1 = jnp.dot(xb_ref[...], w1, preferred_element_type=jnp.float32)
        h1 = jnp.maximum(h1 + b1_ref[0, 0], 0.0)
        w2 = w2_ref[0].astype(jnp.bfloat16)
        eo = jnp.dot(h1.astype(jnp.bfloat16), w2,
                     preferred_element_type=jnp.float32) + b2_ref[0, 0]
        lane = lax.broadcasted_iota(jnp.int32, (B, E), 1)
        gate = jnp.sum(jnp.where(lane == e, g_ref[...], 0.0), axis=1,
                       keepdims=True)
        acc_ref[...] += gate * eo

    @pl.when(i == E - 1)
    def _final():
        out_ref[...] = jnp.dot(acc_ref[...], wf_ref[...],
                               preferred_element_type=jnp.float32) + bf_ref[...]


def kernel(x_list, Wg, bg, W1, b1, W2, b2, Wf, bf):
    x = x_list.reshape(B, D)

    probs, G, idx2d = pl.pallas_call(
        _router_body,
        out_shape=(
            jax.ShapeDtypeStruct((B, E), jnp.float32),
            jax.ShapeDtypeStruct((B, E), jnp.float32),
            jax.ShapeDtypeStruct((E, 1), jnp.int32),
        ),
    )(x, Wg, bg.reshape(1, E))
    idx = idx2d.reshape(E)

    grid_spec = pltpu.PrefetchScalarGridSpec(
        num_scalar_prefetch=1,
        grid=(E,),
        in_specs=[
            pl.BlockSpec((B, D), lambda i, idx_ref: (0, 0)),
            pl.BlockSpec((B, E), lambda i, idx_ref: (0, 0)),
            pl.BlockSpec((1, D, HID), lambda i, idx_ref: (idx_ref[i], 0, 0)),
            pl.BlockSpec((1, 1, HID), lambda i, idx_ref: (idx_ref[i], 0, 0)),
            pl.BlockSpec((1, HID, HID), lambda i, idx_ref: (idx_ref[i], 0, 0)),
            pl.BlockSpec((1, 1, HID), lambda i, idx_ref: (idx_ref[i], 0, 0)),
            pl.BlockSpec((HID, OUT), lambda i, idx_ref: (0, 0)),
            pl.BlockSpec((1, OUT), lambda i, idx_ref: (0, 0)),
        ],
        out_specs=pl.BlockSpec((B, OUT), lambda i, idx_ref: (0, 0)),
        scratch_shapes=[
            pltpu.VMEM((B, HID), jnp.float32),
            pltpu.VMEM((B, D), jnp.bfloat16),
        ],
    )
    out = pl.pallas_call(
        _expert_body,
        grid_spec=grid_spec,
        out_shape=jax.ShapeDtypeStruct((B, OUT), jnp.float32),
    )(idx, x, G, W1, b1.reshape(E, 1, HID), W2, b2.reshape(E, 1, HID),
      Wf, bf.reshape(1, OUT))

    return (out, probs.reshape(1, B, E))
